# baseline (speedup 1.0000x reference)
import jax
import jax.numpy as jnp
from jax.experimental import pallas as pl
from jax.experimental.pallas import tpu as pltpu

T = 7
N = 512
RNN = 256
EMB = 128
OUT = 5
NTS = 32
K = 128
V = 64

BN = 128
NB = N // BN
CH = 128


def _fused(nodes_ref, ttc_ref, ttcv_ref, m_ref,
           win_ref, bin_ref, wt_ref, bt_ref, wtv_ref, btv_ref,
           wih_ref, whh_ref, bg_ref, wout_ref, bout_ref,
           h0_ref, c0_ref,
           out_ref, hs_ref, cs_ref):
    f = pl.program_id(1)

    @pl.when(f == 0)
    def _():
        hs_ref[...] = h0_ref[...]
        cs_ref[...] = c0_ref[...]

    def _colmax(ref, ncols):
        nchunks = ncols // CH
        acc = ref[0, :, 0:CH]

        def body(j, a):
            return jnp.maximum(a, ref[0, :, pl.ds(j * CH, CH)])

        acc = jax.lax.fori_loop(1, nchunks, body, acc)
        a64 = jnp.maximum(acc[:, :64], acc[:, 64:])
        return jnp.maximum(a64[:, :NTS], a64[:, NTS:])

    social = _colmax(ttc_ref, K * NTS)
    social_veh = _colmax(ttcv_ref, V * NTS)

    nodes = nodes_ref[0]
    inp_emb = jax.nn.relu(
        jnp.dot(nodes, win_ref[...], preferred_element_type=jnp.float32)
        + bin_ref[...])
    t_emb = jax.nn.relu(
        jnp.dot(social, wt_ref[...], preferred_element_type=jnp.float32)
        + bt_ref[...])
    tv_emb = jax.nn.relu(
        jnp.dot(social_veh, wtv_ref[...], preferred_element_type=jnp.float32)
        + btv_ref[...])
    concat = jnp.concatenate([inp_emb, t_emb, tv_emb], axis=1)

    h = hs_ref[...]
    c = cs_ref[...]
    gates = (jnp.dot(concat, wih_ref[...], preferred_element_type=jnp.float32)
             + jnp.dot(h, whh_ref[...], preferred_element_type=jnp.float32)
             + bg_ref[...])
    i_g = jax.nn.sigmoid(gates[:, 0:RNN])
    f_g = jax.nn.sigmoid(gates[:, RNN:2 * RNN])
    g_g = jnp.tanh(gates[:, 2 * RNN:3 * RNN])
    o_g = jax.nn.sigmoid(gates[:, 3 * RNN:4 * RNN])
    c_new = f_g * c + i_g * g_g
    h_new = o_g * jnp.tanh(c_new)

    out_raw = (jnp.dot(h_new, wout_ref[...], preferred_element_type=jnp.float32)
               + bout_ref[...])

    m = m_ref[0]
    out_ref[0] = m * out_raw
    hs_ref[...] = h + m * (h_new - h)
    cs_ref[...] = c + m * (c_new - c)


def kernel(input_data, grids, hidden_states, cell_states, mask, input_data_veh,
           grids_veh, mask_veh, grids_TTC, grids_TTC_veh,
           W_in, b_in, W_t, b_t, W_tv, b_tv, W_ih, W_hh, b_ih, b_hh,
           W_out, b_out):
    del grids, input_data_veh, grids_veh, mask_veh

    ttc = grids_TTC.reshape(T, N, K * NTS)
    ttcv = grids_TTC_veh.reshape(T, N, V * NTS)
    maskf = mask.astype(jnp.float32).reshape(T, N, 1)

    win = W_in.T
    wt = W_t.T
    wtv = W_tv.T
    wih = W_ih.T
    whh = W_hh.T
    bg = (b_ih + b_hh).reshape(1, 4 * RNN)
    wout = W_out.T
    bout = b_out.reshape(1, OUT)
    bin2 = b_in.reshape(1, EMB)
    bt2 = b_t.reshape(1, EMB)
    btv2 = b_tv.reshape(1, EMB)

    grid = (NB, T)

    def nb_f(nb, f):
        return (f, nb, 0)

    def const2(nb, f):
        return (0, 0)

    def nb_only(nb, f):
        return (nb, 0)

    outputs, hs, cs = pl.pallas_call(
        _fused,
        grid=grid,
        in_specs=[
            pl.BlockSpec((1, BN, 2), nb_f),
            pl.BlockSpec((1, BN, K * NTS), nb_f),
            pl.BlockSpec((1, BN, V * NTS), nb_f),
            pl.BlockSpec((1, BN, 1), nb_f),
            pl.BlockSpec((2, EMB), const2),
            pl.BlockSpec((1, EMB), const2),
            pl.BlockSpec((NTS, EMB), const2),
            pl.BlockSpec((1, EMB), const2),
            pl.BlockSpec((NTS, EMB), const2),
            pl.BlockSpec((1, EMB), const2),
            pl.BlockSpec((3 * EMB, 4 * RNN), const2),
            pl.BlockSpec((RNN, 4 * RNN), const2),
            pl.BlockSpec((1, 4 * RNN), const2),
            pl.BlockSpec((RNN, OUT), const2),
            pl.BlockSpec((1, OUT), const2),
            pl.BlockSpec((BN, RNN), nb_only),
            pl.BlockSpec((BN, RNN), nb_only),
        ],
        out_specs=[
            pl.BlockSpec((1, BN, OUT), nb_f),
            pl.BlockSpec((BN, RNN), nb_only),
            pl.BlockSpec((BN, RNN), nb_only),
        ],
        out_shape=[
            jax.ShapeDtypeStruct((T, N, OUT), jnp.float32),
            jax.ShapeDtypeStruct((N, RNN), jnp.float32),
            jax.ShapeDtypeStruct((N, RNN), jnp.float32),
        ],
        compiler_params=pltpu.CompilerParams(
            dimension_semantics=("arbitrary", "arbitrary"),
        ),
    )(input_data, ttc, ttcv, maskf,
      win, bin2, wt, bt2, wtv, btv2, wih, whh, bg, wout, bout,
      hidden_states, cell_states)

    return outputs, hs, cs
